# baseline (speedup 1.0000x reference)
import functools
import jax
import jax.numpy as jnp
from jax import lax
from jax.experimental import pallas as pl
from jax.experimental.pallas import tpu as pltpu
from jax.experimental.pallas import tpu_sc as plsc

NC = 2
NS = 16
L = 16
NW = NC * NS
D = 2048
SEQ = 4096
BATCH = 4
NSTRIP = 16
CPW = D // NSTRIP
G = CPW // L
R = 64


def _sc_body(x_hbm, out_hbm, xin_v, out_v):
    wid = lax.axis_index("s") * NC + lax.axis_index("c")
    strip = wid % NSTRIP
    half = wid // NSTRIP
    col0 = strip * CPW
    for bb in range(BATCH // 2):
        b = half * (BATCH // 2) + bb
        zero = (jnp.zeros((L,), jnp.float32),) * G

        @pl.loop(0, SEQ // R, init_carry=zero)
        def _chunk(ci, carry):
            row0 = b * SEQ + ci * R
            pltpu.sync_copy(x_hbm.at[pl.ds(row0, R), pl.ds(col0, CPW)],
                            xin_v)
            carr = list(carry)
            for r in range(R):
                for g in range(G):
                    carr[g] = carr[g] + xin_v[r, pl.ds(g * L, L)]
                    out_v[r, pl.ds(g * L, L)] = carr[g]
            pltpu.sync_copy(out_v,
                            out_hbm.at[pl.ds(row0, R), pl.ds(col0, CPW)])
            return tuple(carr)


def kernel(x):
    B, S, Dm = x.shape
    x2 = x.reshape(B * S, Dm)
    mesh = plsc.VectorSubcoreMesh(core_axis_name="c", subcore_axis_name="s")
    out = pl.kernel(
        _sc_body,
        out_type=jax.ShapeDtypeStruct((B * S, Dm), jnp.float32),
        mesh=mesh,
        scratch_types=[
            pltpu.VMEM((R, CPW), jnp.float32),
            pltpu.VMEM((R, CPW), jnp.float32),
        ],
    )(x2)
    return out.reshape(B, S, Dm)


# speedup vs baseline: 2.8051x; 2.8051x over previous
import jax
import jax.numpy as jnp
from jax.experimental import pallas as pl
from jax.experimental.pallas import tpu as pltpu

S_BLK = 512
D_BLK = 2048
SEQ = 4096


def _scan_body(x_ref, o_ref, carry_ref):
    s = pl.program_id(0)

    @pl.when(s % (SEQ // S_BLK) == 0)
    def _():
        carry_ref[...] = jnp.zeros_like(carry_ref)

    xb = x_ref[...]
    ri = jax.lax.broadcasted_iota(jnp.int32, (S_BLK, S_BLK), 0)
    ci = jax.lax.broadcasted_iota(jnp.int32, (S_BLK, S_BLK), 1)
    tri = (ri >= ci).astype(jnp.float32)
    local = jnp.dot(tri, xb, preferred_element_type=jnp.float32)
    out = local + carry_ref[...]
    o_ref[...] = out
    carry_ref[...] = out[S_BLK - 1:S_BLK, :]


def kernel(x):
    B, S, D = x.shape
    x2 = x.reshape(B * S, D)
    out = pl.pallas_call(
        _scan_body,
        grid=(B * S // S_BLK,),
        in_specs=[pl.BlockSpec((S_BLK, D_BLK), lambda s: (s, 0))],
        out_specs=pl.BlockSpec((S_BLK, D_BLK), lambda s: (s, 0)),
        out_shape=jax.ShapeDtypeStruct(x2.shape, x2.dtype),
        scratch_shapes=[pltpu.VMEM((1, D_BLK), jnp.float32)],
        compiler_params=pltpu.CompilerParams(
            dimension_semantics=("arbitrary",)),
    )(x2)
    return out.reshape(B, S, D)
